# Initial kernel scaffold; baseline (speedup 1.0000x reference)
import jax
import jax.numpy as jnp
from jax.experimental import pallas as pl


def kernel(x, adj, W1, b1, W2, b2):
    raise NotImplementedError("write your pallas kernel here")


# speedup vs baseline: 7.4099x; 7.4099x over previous
import functools

import jax
import jax.numpy as jnp
from jax import lax
from jax.experimental import pallas as pl
from jax.experimental.pallas import tpu as pltpu
from jax.experimental.pallas import tpu_sc as plsc

_N, _K, _D = 10000, 32, 128
_BN = 80
_NB = _N // _BN
_NC, _NS = 2, 16
_NW = _NC * _NS


def _gather_sum_kernel(table_hbm, adjb_hbm, out_hbm, idx_v, acc_v, sem):
    wid = lax.axis_index("s") * _NC + lax.axis_index("c")

    def blk(t, carry):
        b = wid + t * _NW

        @pl.when(b < _NB)
        def _():
            pltpu.sync_copy(adjb_hbm.at[b], idx_v)
            pltpu.async_copy(table_hbm.at[idx_v.at[0]], acc_v, sem).wait()

            def fire(k, c):
                pltpu.async_copy(
                    table_hbm.at[idx_v.at[k]], acc_v, sem, add=True)
                return c

            lax.fori_loop(1, _K, fire, 0)

            def drain(k, c):
                pltpu.make_async_copy(
                    table_hbm.at[idx_v.at[0]], acc_v, sem).wait()
                return c

            lax.fori_loop(1, _K, drain, 0)
            pltpu.sync_copy(acc_v, out_hbm.at[pl.ds(b * _BN, _BN)])

        return carry

    nblk_max = (_NB + _NW - 1) // _NW
    lax.fori_loop(0, nblk_max, blk, 0)


@functools.cache
def _build_gather_sum():
    return pl.kernel(
        _gather_sum_kernel,
        out_type=jax.ShapeDtypeStruct((_N, _D), jnp.float32),
        mesh=plsc.VectorSubcoreMesh(
            core_axis_name="c", subcore_axis_name="s",
            num_cores=_NC, num_subcores=_NS),
        scratch_types=[
            pltpu.VMEM((_K, _BN), jnp.int32),
            pltpu.VMEM((_BN, _D), jnp.float32),
            pltpu.SemaphoreType.DMA,
        ],
    )


def _dense(x, s, wx, wa, b, *, final):
    bm = 400
    grid = (_N // bm,)

    def body(x_ref, s_ref, wx_ref, wa_ref, b_ref, o_ref):
        z = jnp.dot(x_ref[...], wx_ref[...], preferred_element_type=jnp.float32)
        z = z + jnp.dot(s_ref[...], wa_ref[...],
                        preferred_element_type=jnp.float32)
        z = z + b_ref[...]
        if final:
            m = jnp.max(z, axis=1, keepdims=True)
            e = jnp.exp(z - m)
            o_ref[...] = z - m - jnp.log(jnp.sum(e, axis=1, keepdims=True))
        else:
            o_ref[...] = jnp.maximum(z, 0.0)

    d = x.shape[1]
    h = wx.shape[1]
    return pl.pallas_call(
        body,
        grid=grid,
        in_specs=[
            pl.BlockSpec((bm, d), lambda i: (i, 0)),
            pl.BlockSpec((bm, d), lambda i: (i, 0)),
            pl.BlockSpec((d, h), lambda i: (0, 0)),
            pl.BlockSpec((d, h), lambda i: (0, 0)),
            pl.BlockSpec((1, h), lambda i: (0, 0)),
        ],
        out_specs=pl.BlockSpec((bm, h), lambda i: (i, 0)),
        out_shape=jax.ShapeDtypeStruct((_N, h), jnp.float32),
    )(x, s, wx, wa, b)


def kernel(x, adj, W1, b1, W2, b2):
    adjb = adj.reshape(_NB, _BN, _K).transpose(0, 2, 1)
    d = x.shape[1]
    wx1, wa1 = W1[:, :d].T, W1[:, d:].T / _K
    h = W1.shape[0]
    wx2, wa2 = W2[:, :h].T, W2[:, h:].T / _K

    gather_sum = _build_gather_sum()
    s1 = gather_sum(x, adjb)
    h1 = _dense(x, s1, wx1, wa1, b1.reshape(1, -1), final=False)
    s2 = gather_sum(h1, adjb)
    return _dense(h1, s2, wx2, wa2, b2.reshape(1, -1), final=True)


# speedup vs baseline: 7.9789x; 1.0768x over previous
import functools

import jax
import jax.numpy as jnp
from jax import lax
from jax.experimental import pallas as pl
from jax.experimental.pallas import tpu as pltpu
from jax.experimental.pallas import tpu_sc as plsc

_N, _K, _D = 10000, 32, 128
_BN = 80
_NB = _N // _BN
_NC, _NS = 2, 16
_NW = _NC * _NS


def _gather_sum_kernel(table_hbm, adjb_hbm, out_hbm, tbl_sh, idx_v, acc_v,
                       sem):
    sid = lax.axis_index("s")
    wid = sid * _NC + lax.axis_index("c")

    @pl.when(sid < _NS - 1)
    def _():
        pltpu.sync_copy(table_hbm.at[pl.ds(sid * 624, 624)],
                        tbl_sh.at[pl.ds(sid * 624, 624)])

    @pl.when(sid == _NS - 1)
    def _():
        pltpu.sync_copy(table_hbm.at[pl.ds(624 * (_NS - 1), _N - 624 * (_NS - 1))],
                        tbl_sh.at[pl.ds(624 * (_NS - 1), _N - 624 * (_NS - 1))])

    plsc.subcore_barrier()

    def blk(t, carry):
        b = wid + t * _NW

        @pl.when(b < _NB)
        def _():
            pltpu.sync_copy(adjb_hbm.at[b], idx_v)
            pltpu.async_copy(tbl_sh.at[idx_v.at[0]], acc_v, sem).wait()

            def fire(k, c):
                pltpu.async_copy(
                    tbl_sh.at[idx_v.at[k]], acc_v, sem, add=True)
                return c

            lax.fori_loop(1, _K, fire, 0)

            def drain(k, c):
                pltpu.make_async_copy(
                    table_hbm.at[idx_v.at[0]], acc_v, sem).wait()
                return c

            lax.fori_loop(1, _K, drain, 0)
            pltpu.sync_copy(acc_v, out_hbm.at[pl.ds(b * _BN, _BN)])

        return carry

    nblk_max = (_NB + _NW - 1) // _NW
    lax.fori_loop(0, nblk_max, blk, 0)


@functools.cache
def _build_gather_sum():
    return pl.kernel(
        _gather_sum_kernel,
        out_type=jax.ShapeDtypeStruct((_N, _D), jnp.float32),
        mesh=plsc.VectorSubcoreMesh(
            core_axis_name="c", subcore_axis_name="s",
            num_cores=_NC, num_subcores=_NS),
        scratch_types=[
            pltpu.VMEM_SHARED((_N, _D), jnp.float32),
            pltpu.VMEM((_K, _BN), jnp.int32),
            pltpu.VMEM((_BN, _D), jnp.float32),
            pltpu.SemaphoreType.DMA,
        ],
    )


def _dense(x, s, wx, wa, b, *, final):
    bm = 400
    grid = (_N // bm,)

    def body(x_ref, s_ref, wx_ref, wa_ref, b_ref, o_ref):
        z = jnp.dot(x_ref[...], wx_ref[...], preferred_element_type=jnp.float32)
        z = z + jnp.dot(s_ref[...], wa_ref[...],
                        preferred_element_type=jnp.float32)
        z = z + b_ref[...]
        if final:
            m = jnp.max(z, axis=1, keepdims=True)
            e = jnp.exp(z - m)
            o_ref[...] = z - m - jnp.log(jnp.sum(e, axis=1, keepdims=True))
        else:
            o_ref[...] = jnp.maximum(z, 0.0)

    d = x.shape[1]
    h = wx.shape[1]
    return pl.pallas_call(
        body,
        grid=grid,
        in_specs=[
            pl.BlockSpec((bm, d), lambda i: (i, 0)),
            pl.BlockSpec((bm, d), lambda i: (i, 0)),
            pl.BlockSpec((d, h), lambda i: (0, 0)),
            pl.BlockSpec((d, h), lambda i: (0, 0)),
            pl.BlockSpec((1, h), lambda i: (0, 0)),
        ],
        out_specs=pl.BlockSpec((bm, h), lambda i: (i, 0)),
        out_shape=jax.ShapeDtypeStruct((_N, h), jnp.float32),
    )(x, s, wx, wa, b)


def kernel(x, adj, W1, b1, W2, b2):
    adjb = adj.reshape(_NB, _BN, _K).transpose(0, 2, 1)
    d = x.shape[1]
    wx1, wa1 = W1[:, :d].T, W1[:, d:].T / _K
    h = W1.shape[0]
    wx2, wa2 = W2[:, :h].T, W2[:, h:].T / _K

    gather_sum = _build_gather_sum()
    s1 = gather_sum(x, adjb)
    h1 = _dense(x, s1, wx1, wa1, b1.reshape(1, -1), final=False)
    s2 = gather_sum(h1, adjb)
    return _dense(h1, s2, wx2, wa2, b2.reshape(1, -1), final=True)
